# baseline (speedup 1.0000x reference)
import functools

import jax
import jax.numpy as jnp
from jax import lax
from jax.experimental import pallas as pl
from jax.experimental.pallas import tpu as pltpu
from jax.experimental.pallas import tpu_sc as plsc

F = 26
CHUNK = 128
N_PIECES = 8


def _build_sc_call(B, V):
    info = plsc.get_sparse_core_info()
    NC, NS = info.num_cores, info.num_subcores
    NW = NC * NS
    b_per_w = B // NW
    idx_per_w = b_per_w * F
    n_chunks = idx_per_w // CHUNK
    piece = V // N_PIECES
    assert idx_per_w % CHUNK == 0 and b_per_w % 16 == 0
    assert V % N_PIECES == 0 and piece % 8 == 0

    mesh = plsc.VectorSubcoreMesh(core_axis_name="c", subcore_axis_name="s")

    @functools.partial(
        pl.kernel,
        mesh=mesh,
        compiler_params=pltpu.CompilerParams(
            use_tc_tiling_on_sc=False, needs_layout_passes=False
        ),
        out_type=jax.ShapeDtypeStruct((B,), jnp.float32),
        scratch_types=[
            pltpu.VMEM_SHARED((1, V), jnp.float32),
            pltpu.VMEM((n_chunks, CHUNK), jnp.int32),
            pltpu.VMEM((idx_per_w,), jnp.float32),
            pltpu.VMEM((16,), jnp.float32),
            pltpu.VMEM((b_per_w,), jnp.float32),
            pltpu.SemaphoreType.DMA((F,)),
        ],
    )
    def sc_kernel(x_hbm, *rest):
        piece_refs = rest[:N_PIECES]
        bias_hbm = rest[N_PIECES]
        out_hbm = rest[N_PIECES + 1]
        tab_sh, idx_v, vals_v, bias_v, out_v, sem = rest[N_PIECES + 2:]

        cc = lax.axis_index("c")
        ss = lax.axis_index("s")
        wid = ss * NC + cc

        half = (piece // 2) & ~7
        for p in range(N_PIECES):
            @pl.when(ss == 2 * p)
            def _(p=p):
                pltpu.sync_copy(
                    piece_refs[p].at[:, pl.ds(0, half)],
                    tab_sh.at[:, pl.ds(p * piece, half)],
                )

            @pl.when(ss == 2 * p + 1)
            def _(p=p):
                pltpu.sync_copy(
                    piece_refs[p].at[:, pl.ds(half, piece - half)],
                    tab_sh.at[:, pl.ds(p * piece + half, piece - half)],
                )

        pltpu.sync_copy(x_hbm.at[wid], idx_v)
        pltpu.sync_copy(bias_hbm, bias_v)

        plsc.subcore_barrier()

        tab_flat = tab_sh.at[0]
        chunks_per_f = b_per_w // CHUNK

        def start_chunk(j, carry):
            pltpu.make_async_copy(
                tab_flat.at[idx_v.at[j]],
                vals_v.at[pl.ds(j * CHUNK, CHUNK)],
                sem.at[j // chunks_per_f],
            ).start()
            return carry

        lax.fori_loop(0, n_chunks, start_chunk, 0)

        bias_vec = bias_v[...]

        accs = [bias_vec] * (b_per_w // 16)
        for f in range(F):
            pltpu.make_async_copy(
                tab_flat.at[pl.ds(0, b_per_w)],
                vals_v.at[pl.ds(f * b_per_w, b_per_w)],
                sem.at[f],
            ).wait()
            for jb in range(b_per_w // 16):
                accs[jb] = accs[jb] + vals_v[pl.ds(f * b_per_w + jb * 16, 16)]
        for jb in range(b_per_w // 16):
            out_v[pl.ds(jb * 16, 16)] = accs[jb]

        pltpu.sync_copy(out_v, out_hbm.at[pl.ds(wid * b_per_w, b_per_w)])

    return sc_kernel, NW


def kernel(X, table, bias):
    B, f = X.shape
    V = table.shape[0]
    assert f == F
    sc_call, NW = _build_sc_call(B, V)
    b_per_w = B // NW
    x_blocks = (
        X.astype(jnp.int32)
        .reshape(NW, b_per_w, F)
        .transpose(0, 2, 1)
        .reshape(NW, b_per_w * F // CHUNK, CHUNK)
    )
    c = V // N_PIECES
    tf = table.astype(jnp.float32)
    pieces = [tf[i * c:(i + 1) * c, 0].reshape(1, c) for i in range(N_PIECES)]
    bias16 = jnp.broadcast_to(bias.astype(jnp.float32), (16,))
    y = sc_call(x_blocks, *pieces, bias16)
    return y.reshape(B, 1)


# speedup vs baseline: 1.0473x; 1.0473x over previous
import functools

import jax
import jax.numpy as jnp
from jax import lax
from jax.experimental import pallas as pl
from jax.experimental.pallas import tpu as pltpu
from jax.experimental.pallas import tpu_sc as plsc

F = 26
CHUNK = 128
N_PIECES = 8


def _build_sc_call(B, V):
    info = plsc.get_sparse_core_info()
    NC, NS = info.num_cores, info.num_subcores
    NW = NC * NS
    b_per_w = B // NW
    idx_per_w = b_per_w * F
    n_chunks = idx_per_w // CHUNK
    piece = V // N_PIECES
    assert idx_per_w % CHUNK == 0 and b_per_w % 16 == 0
    assert V % N_PIECES == 0 and piece % 8 == 0

    mesh = plsc.VectorSubcoreMesh(core_axis_name="c", subcore_axis_name="s")

    @functools.partial(
        pl.kernel,
        mesh=mesh,
        compiler_params=pltpu.CompilerParams(
            use_tc_tiling_on_sc=False, needs_layout_passes=False
        ),
        out_type=jax.ShapeDtypeStruct((B,), jnp.float32),
        scratch_types=[
            pltpu.VMEM_SHARED((1, V), jnp.float32),
            pltpu.VMEM((n_chunks, CHUNK), jnp.int32),
            pltpu.VMEM((idx_per_w,), jnp.float32),
            pltpu.VMEM((16,), jnp.float32),
            pltpu.VMEM((b_per_w,), jnp.float32),
            pltpu.SemaphoreType.DMA,
        ],
    )
    def sc_kernel(x_hbm, *rest):
        piece_refs = rest[:N_PIECES]
        bias_hbm = rest[N_PIECES]
        out_hbm = rest[N_PIECES + 1]
        tab_sh, idx_v, vals_v, bias_v, out_v, sem = rest[N_PIECES + 2:]

        cc = lax.axis_index("c")
        ss = lax.axis_index("s")
        wid = ss * NC + cc

        half = (piece // 2) & ~7
        for p in range(N_PIECES):
            @pl.when(ss == 2 * p)
            def _(p=p):
                pltpu.sync_copy(
                    piece_refs[p].at[:, pl.ds(0, half)],
                    tab_sh.at[:, pl.ds(p * piece, half)],
                )

            @pl.when(ss == 2 * p + 1)
            def _(p=p):
                pltpu.sync_copy(
                    piece_refs[p].at[:, pl.ds(half, piece - half)],
                    tab_sh.at[:, pl.ds(p * piece + half, piece - half)],
                )

        pltpu.sync_copy(x_hbm.at[wid], idx_v)
        pltpu.sync_copy(bias_hbm, bias_v)

        plsc.subcore_barrier()

        tab_flat = tab_sh.at[0]

        def start_chunks(q, carry):
            for u in range(4):
                j = q * 4 + u
                pltpu.make_async_copy(
                    tab_flat.at[idx_v.at[j]],
                    vals_v.at[pl.ds(j * CHUNK, CHUNK)],
                    sem,
                ).start()
            return carry

        lax.fori_loop(0, n_chunks // 4, start_chunks, 0)

        pltpu.make_async_copy(
            tab_flat.at[pl.ds(0, idx_per_w)], vals_v, sem
        ).wait()

        bias_vec = bias_v[...]

        def row_block(jb, carry):
            base = jb * 16
            acc = bias_vec
            for f in range(F):
                acc = acc + vals_v[pl.ds(f * b_per_w + base, 16)]
            out_v[pl.ds(base, 16)] = acc
            return carry

        lax.fori_loop(0, b_per_w // 16, row_block, 0)

        pltpu.sync_copy(out_v, out_hbm.at[pl.ds(wid * b_per_w, b_per_w)])

    return sc_kernel, NW


def kernel(X, table, bias):
    B, f = X.shape
    V = table.shape[0]
    assert f == F
    sc_call, NW = _build_sc_call(B, V)
    b_per_w = B // NW
    x_blocks = (
        X.astype(jnp.int32)
        .reshape(NW, b_per_w, F)
        .transpose(0, 2, 1)
        .reshape(NW, b_per_w * F // CHUNK, CHUNK)
    )
    c = V // N_PIECES
    tf = table.astype(jnp.float32)
    pieces = [tf[i * c:(i + 1) * c, 0].reshape(1, c) for i in range(N_PIECES)]
    bias16 = jnp.broadcast_to(bias.astype(jnp.float32), (16,))
    y = sc_call(x_blocks, *pieces, bias16)
    return y.reshape(B, 1)


# speedup vs baseline: 1.0968x; 1.0473x over previous
import functools

import jax
import jax.numpy as jnp
from jax import lax
from jax.experimental import pallas as pl
from jax.experimental.pallas import tpu as pltpu
from jax.experimental.pallas import tpu_sc as plsc

F = 26
CHUNK = 128
N_PIECES = 16


def _piece_bounds(V):
    bounds = [(V * i // N_PIECES) & ~7 for i in range(N_PIECES)] + [V]
    return bounds


def _build_sc_call(B, V):
    info = plsc.get_sparse_core_info()
    NC, NS = info.num_cores, info.num_subcores
    NW = NC * NS
    b_per_w = B // NW
    idx_per_w = b_per_w * F
    n_chunks = idx_per_w // CHUNK
    bounds = _piece_bounds(V)
    assert idx_per_w % CHUNK == 0 and b_per_w % 16 == 0

    mesh = plsc.VectorSubcoreMesh(core_axis_name="c", subcore_axis_name="s")

    @functools.partial(
        pl.kernel,
        mesh=mesh,
        compiler_params=pltpu.CompilerParams(
            use_tc_tiling_on_sc=False, needs_layout_passes=False
        ),
        out_type=jax.ShapeDtypeStruct((B,), jnp.float32),
        scratch_types=[
            pltpu.VMEM_SHARED((1, V), jnp.float32),
            pltpu.VMEM((n_chunks, CHUNK), jnp.int32),
            pltpu.VMEM((idx_per_w,), jnp.float32),
            pltpu.VMEM((16,), jnp.float32),
            pltpu.VMEM((b_per_w,), jnp.float32),
            pltpu.SemaphoreType.DMA,
        ],
    )
    def sc_kernel(x_hbm, *rest):
        piece_refs = rest[:N_PIECES]
        bias_hbm = rest[N_PIECES]
        out_hbm = rest[N_PIECES + 1]
        tab_sh, idx_v, vals_v, bias_v, out_v, sem = rest[N_PIECES + 2:]

        cc = lax.axis_index("c")
        ss = lax.axis_index("s")
        wid = ss * NC + cc

        for p in range(N_PIECES):
            @pl.when(ss == p)
            def _(p=p):
                pltpu.sync_copy(
                    piece_refs[p],
                    tab_sh.at[:, pl.ds(bounds[p], bounds[p + 1] - bounds[p])],
                )

        pltpu.sync_copy(x_hbm.at[wid], idx_v)
        pltpu.sync_copy(bias_hbm, bias_v)

        plsc.subcore_barrier()

        tab_flat = tab_sh.at[0]

        def start_chunks(q, carry):
            for u in range(4):
                j = q * 4 + u
                pltpu.make_async_copy(
                    tab_flat.at[idx_v.at[j]],
                    vals_v.at[pl.ds(j * CHUNK, CHUNK)],
                    sem,
                ).start()
            return carry

        lax.fori_loop(0, n_chunks // 4, start_chunks, 0)

        pltpu.make_async_copy(
            tab_flat.at[pl.ds(0, idx_per_w)], vals_v, sem
        ).wait()

        bias_vec = bias_v[...]

        def row_block(jb, carry):
            base = jb * 16
            acc = bias_vec
            for f in range(F):
                acc = acc + vals_v[pl.ds(f * b_per_w + base, 16)]
            out_v[pl.ds(base, 16)] = acc
            return carry

        lax.fori_loop(0, b_per_w // 16, row_block, 0)

        pltpu.sync_copy(out_v, out_hbm.at[pl.ds(wid * b_per_w, b_per_w)])

    return sc_kernel, NW


def kernel(X, table, bias):
    B, f = X.shape
    V = table.shape[0]
    assert f == F
    sc_call, NW = _build_sc_call(B, V)
    b_per_w = B // NW
    x_blocks = (
        X.astype(jnp.int32)
        .reshape(NW, b_per_w, F)
        .transpose(0, 2, 1)
        .reshape(NW, b_per_w * F // CHUNK, CHUNK)
    )
    bounds = _piece_bounds(V)
    tf = table.astype(jnp.float32)
    pieces = [
        tf[bounds[i]:bounds[i + 1], 0].reshape(1, bounds[i + 1] - bounds[i])
        for i in range(N_PIECES)
    ]
    bias16 = jnp.broadcast_to(bias.astype(jnp.float32), (16,))
    y = sc_call(x_blocks, *pieces, bias16)
    return y.reshape(B, 1)
